# baseline (speedup 1.0000x reference)
import functools

import jax
import jax.numpy as jnp
from jax.experimental import pallas as pl
from jax.experimental.pallas import tpu as pltpu

TOP_K = 8
BT = 2048
RC = 128


def _router_block(x_ref, w1_ref, b1_ref, w2_ref, b2_ref, w3_ref, b3_ref,
                  idx_ref, wgt_ref):
    x = x_ref[...]
    h = jnp.dot(x, w1_ref[...], preferred_element_type=jnp.float32)
    h = jnp.maximum(h + b1_ref[...], 0.0)
    h = jnp.dot(h, w2_ref[...], preferred_element_type=jnp.float32)
    h = jnp.maximum(h + b2_ref[...], 0.0)
    s = jnp.dot(h, w3_ref[...], preferred_element_type=jnp.float32)
    s = s + b3_ref[...]

    num_e = s.shape[-1]
    rc = RC
    flane = jax.lax.broadcasted_iota(
        jnp.int32, (rc, num_e), 1).astype(jnp.float32)
    for c in range(BT // rc):
        sc = s[c * rc:(c + 1) * rc, :]
        vals = []
        idxs = []
        for _ in range(TOP_K):
            mx = jnp.max(sc, axis=1, keepdims=True)
            imf = jnp.min(jnp.where(sc == mx, flane, jnp.float32(num_e)),
                          axis=1, keepdims=True)
            vals.append(mx)
            idxs.append(imf)
            sc = jnp.where(flane == imf, -jnp.inf, sc)
        v = jnp.concatenate(vals, axis=1)
        i32 = jnp.concatenate(idxs, axis=1).astype(jnp.int32)
        e = jnp.exp(v - v[:, :1])
        w = e / jnp.sum(e, axis=1, keepdims=True)
        idx_ref[c * rc:(c + 1) * rc, :] = i32
        wgt_ref[c * rc:(c + 1) * rc, :] = w


@jax.jit
def _run(x, w1, b1, w2, b2, w3, b3):
    b, d = x.shape
    h = w1.shape[1]
    e = w3.shape[1]
    grid = (b // BT,)
    return pl.pallas_call(
        _router_block,
        grid=grid,
        in_specs=[
            pl.BlockSpec((BT, d), lambda i: (i, 0)),
            pl.BlockSpec((d, h), lambda i: (0, 0)),
            pl.BlockSpec((1, h), lambda i: (0, 0)),
            pl.BlockSpec((h, h), lambda i: (0, 0)),
            pl.BlockSpec((1, h), lambda i: (0, 0)),
            pl.BlockSpec((h, e), lambda i: (0, 0)),
            pl.BlockSpec((1, e), lambda i: (0, 0)),
        ],
        out_specs=[
            pl.BlockSpec((BT, TOP_K), lambda i: (i, 0)),
            pl.BlockSpec((BT, TOP_K), lambda i: (i, 0)),
        ],
        out_shape=[
            jax.ShapeDtypeStruct((b, TOP_K), jnp.int32),
            jax.ShapeDtypeStruct((b, TOP_K), jnp.float32),
        ],
    )(x, w1, b1, w2, b2, w3, b3)


def kernel(prompt_embedding, W1, b1, W2, b2, W3, b3):
    idx, wgt = _run(prompt_embedding.astype(jnp.float32), W1,
                    b1.reshape(1, -1), W2, b2.reshape(1, -1), W3,
                    b3.reshape(1, -1))
    return idx, wgt, idx[:, 0]


# speedup vs baseline: 2.3743x; 2.3743x over previous
import functools

import jax
import jax.numpy as jnp
from jax.experimental import pallas as pl
from jax.experimental.pallas import tpu as pltpu

TOP_K = 8
BT = 1024


def _router_block(x_ref, w1t_ref, b1_ref, w2t_ref, b2_ref, w3t_ref, b3_ref,
                  idx_ref, wgt_ref):
    x = x_ref[...]
    h = jax.lax.dot_general(w1t_ref[...], x, (((1,), (1,)), ((), ())),
                            preferred_element_type=jnp.float32)
    h = jnp.maximum(h + b1_ref[...], 0.0)
    h = jax.lax.dot_general(w2t_ref[...], h, (((1,), (0,)), ((), ())),
                            preferred_element_type=jnp.float32)
    h = jnp.maximum(h + b2_ref[...], 0.0)
    s = jax.lax.dot_general(w3t_ref[...], h, (((1,), (0,)), ((), ())),
                            preferred_element_type=jnp.float32)
    s = s + b3_ref[...]

    num_e = s.shape[0]
    flane = jax.lax.broadcasted_iota(jnp.int32, s.shape, 0).astype(jnp.float32)
    vals = []
    idxs = []
    for _ in range(TOP_K):
        mx = jnp.max(s, axis=0, keepdims=True)
        imf = jnp.min(jnp.where(s == mx, flane, jnp.float32(num_e)),
                      axis=0, keepdims=True)
        vals.append(mx)
        idxs.append(imf)
        s = jnp.where(flane == imf, -jnp.inf, s)
    v = jnp.concatenate(vals, axis=0)
    im = jnp.concatenate(idxs, axis=0)
    e = jnp.exp(v - v[:1])
    w = e / jnp.sum(e, axis=0, keepdims=True)
    idx_ref[...] = im.astype(jnp.int32).T
    wgt_ref[...] = w.T


@jax.jit
def _run(x, w1t, b1, w2t, b2, w3t, b3):
    b, d = x.shape
    h = w1t.shape[0]
    e = w3t.shape[0]
    grid = (b // BT,)
    return pl.pallas_call(
        _router_block,
        grid=grid,
        in_specs=[
            pl.BlockSpec((BT, d), lambda i: (i, 0)),
            pl.BlockSpec((h, d), lambda i: (0, 0)),
            pl.BlockSpec((h, 1), lambda i: (0, 0)),
            pl.BlockSpec((h, h), lambda i: (0, 0)),
            pl.BlockSpec((h, 1), lambda i: (0, 0)),
            pl.BlockSpec((e, h), lambda i: (0, 0)),
            pl.BlockSpec((e, 1), lambda i: (0, 0)),
        ],
        out_specs=[
            pl.BlockSpec((BT, TOP_K), lambda i: (i, 0)),
            pl.BlockSpec((BT, TOP_K), lambda i: (i, 0)),
        ],
        out_shape=[
            jax.ShapeDtypeStruct((b, TOP_K), jnp.int32),
            jax.ShapeDtypeStruct((b, TOP_K), jnp.float32),
        ],
    )(x, w1t, b1, w2t, b2, w3t, b3)


def kernel(prompt_embedding, W1, b1, W2, b2, W3, b3):
    idx, wgt = _run(prompt_embedding.astype(jnp.float32), W1.T,
                    b1.reshape(-1, 1), W2.T, b2.reshape(-1, 1), W3.T,
                    b3.reshape(-1, 1))
    return idx, wgt, idx[:, 0]


# speedup vs baseline: 2.6687x; 1.1240x over previous
import functools

import jax
import jax.numpy as jnp
from jax.experimental import pallas as pl
from jax.experimental.pallas import tpu as pltpu

TOP_K = 8
BT = 2048


def _router_block(x_ref, w1t_ref, b1_ref, w2t_ref, b2_ref, w3t_ref, b3_ref,
                  idx_ref, wgt_ref):
    x = x_ref[...]
    h = jax.lax.dot_general(w1t_ref[...], x, (((1,), (1,)), ((), ())),
                            preferred_element_type=jnp.float32)
    h = jnp.maximum(h + b1_ref[...], 0.0)
    h = jax.lax.dot_general(w2t_ref[...], h, (((1,), (0,)), ((), ())),
                            preferred_element_type=jnp.float32)
    h = jnp.maximum(h + b2_ref[...], 0.0)
    s = jax.lax.dot_general(w3t_ref[...], h, (((1,), (0,)), ((), ())),
                            preferred_element_type=jnp.float32)
    s = s + b3_ref[...]

    num_e = s.shape[0]
    flane = jax.lax.broadcasted_iota(jnp.int32, s.shape, 0).astype(jnp.float32)
    vals = []
    idxs = []
    for _ in range(TOP_K):
        mx = jnp.max(s, axis=0, keepdims=True)
        imf = jnp.min(jnp.where(s == mx, flane, jnp.float32(num_e)),
                      axis=0, keepdims=True)
        vals.append(mx)
        idxs.append(imf)
        s = jnp.where(flane == imf, -jnp.inf, s)
    v = jnp.concatenate(vals, axis=0)
    im = jnp.concatenate(idxs, axis=0)
    e = jnp.exp(v - v[:1])
    w = e / jnp.sum(e, axis=0, keepdims=True)
    idx_ref[...] = im.astype(jnp.int32).T
    wgt_ref[...] = w.T


@jax.jit
def _run(x, w1t, b1, w2t, b2, w3t, b3):
    b, d = x.shape
    h = w1t.shape[0]
    e = w3t.shape[0]
    grid = (b // BT,)
    return pl.pallas_call(
        _router_block,
        grid=grid,
        in_specs=[
            pl.BlockSpec((BT, d), lambda i: (i, 0)),
            pl.BlockSpec((h, d), lambda i: (0, 0)),
            pl.BlockSpec((h, 1), lambda i: (0, 0)),
            pl.BlockSpec((h, h), lambda i: (0, 0)),
            pl.BlockSpec((h, 1), lambda i: (0, 0)),
            pl.BlockSpec((e, h), lambda i: (0, 0)),
            pl.BlockSpec((e, 1), lambda i: (0, 0)),
        ],
        out_specs=[
            pl.BlockSpec((BT, TOP_K), lambda i: (i, 0)),
            pl.BlockSpec((BT, TOP_K), lambda i: (i, 0)),
        ],
        out_shape=[
            jax.ShapeDtypeStruct((b, TOP_K), jnp.int32),
            jax.ShapeDtypeStruct((b, TOP_K), jnp.float32),
        ],
    )(x, w1t, b1, w2t, b2, w3t, b3)


def kernel(prompt_embedding, W1, b1, W2, b2, W3, b3):
    idx, wgt = _run(prompt_embedding.astype(jnp.float32), W1.T,
                    b1.reshape(-1, 1), W2.T, b2.reshape(-1, 1), W3.T,
                    b3.reshape(-1, 1))
    return idx, wgt, idx[:, 0]


# speedup vs baseline: 2.7922x; 1.0463x over previous
import functools

import jax
import jax.numpy as jnp
from jax.experimental import pallas as pl
from jax.experimental.pallas import tpu as pltpu

TOP_K = 8
BT = 4096


def _router_block(x_ref, w1t_ref, b1_ref, w2t_ref, b2_ref, w3t_ref, b3_ref,
                  idx_ref, wgt_ref):
    x = x_ref[...]
    h = jax.lax.dot_general(w1t_ref[...], x, (((1,), (1,)), ((), ())),
                            preferred_element_type=jnp.float32)
    h = jnp.maximum(h + b1_ref[...], 0.0)
    h = jax.lax.dot_general(w2t_ref[...], h, (((1,), (0,)), ((), ())),
                            preferred_element_type=jnp.float32)
    h = jnp.maximum(h + b2_ref[...], 0.0)
    s = jax.lax.dot_general(w3t_ref[...], h, (((1,), (0,)), ((), ())),
                            preferred_element_type=jnp.float32)
    s = s + b3_ref[...]

    num_e = s.shape[0]
    flane = jax.lax.broadcasted_iota(jnp.int32, s.shape, 0).astype(jnp.float32)
    vals = []
    idxs = []
    for _ in range(TOP_K):
        mx = jnp.max(s, axis=0, keepdims=True)
        imf = jnp.min(jnp.where(s == mx, flane, jnp.float32(num_e)),
                      axis=0, keepdims=True)
        vals.append(mx)
        idxs.append(imf)
        s = jnp.where(flane == imf, -jnp.inf, s)
    v = jnp.concatenate(vals, axis=0)
    im = jnp.concatenate(idxs, axis=0)
    e = jnp.exp(v - v[:1])
    w = e / jnp.sum(e, axis=0, keepdims=True)
    idx_ref[...] = im.astype(jnp.int32).T
    wgt_ref[...] = w.T


@jax.jit
def _run(x, w1t, b1, w2t, b2, w3t, b3):
    b, d = x.shape
    h = w1t.shape[0]
    e = w3t.shape[0]
    grid = (b // BT,)
    return pl.pallas_call(
        _router_block,
        grid=grid,
        in_specs=[
            pl.BlockSpec((BT, d), lambda i: (i, 0)),
            pl.BlockSpec((h, d), lambda i: (0, 0)),
            pl.BlockSpec((h, 1), lambda i: (0, 0)),
            pl.BlockSpec((h, h), lambda i: (0, 0)),
            pl.BlockSpec((h, 1), lambda i: (0, 0)),
            pl.BlockSpec((e, h), lambda i: (0, 0)),
            pl.BlockSpec((e, 1), lambda i: (0, 0)),
        ],
        out_specs=[
            pl.BlockSpec((BT, TOP_K), lambda i: (i, 0)),
            pl.BlockSpec((BT, TOP_K), lambda i: (i, 0)),
        ],
        out_shape=[
            jax.ShapeDtypeStruct((b, TOP_K), jnp.int32),
            jax.ShapeDtypeStruct((b, TOP_K), jnp.float32),
        ],
    )(x, w1t, b1, w2t, b2, w3t, b3)


def kernel(prompt_embedding, W1, b1, W2, b2, W3, b3):
    idx, wgt = _run(prompt_embedding.astype(jnp.float32), W1.T,
                    b1.reshape(-1, 1), W2.T, b2.reshape(-1, 1), W3.T,
                    b3.reshape(-1, 1))
    return idx, wgt, idx[:, 0]
